# baseline (speedup 1.0000x reference)
import functools

import jax
import jax.numpy as jnp
from jax.experimental import pallas as pl
from jax.experimental.pallas import tpu as pltpu

_MEAN = jnp.array([0.485, 0.456, 0.406], jnp.float32)
_STD = jnp.array([0.229, 0.224, 0.225], jnp.float32)

_VMEM_LIMIT = 48 * 1024 * 1024


def _round_up(x, m):
    return (x + m - 1) // m * m


def _pick_div(n, cap):
    for t in range(min(n, cap), 0, -1):
        if n % t == 0:
            return t
    return 1


def _conv_kernel(xa_ref, xb_ref, w_ref, b_ref, o_ref, *, TH, W, Wpad, fold_dy):
    Cin = xa_ref.shape[-1]
    Cout = o_ref.shape[-1]
    M = TH * Wpad

    a = xa_ref[0].reshape(M, Cin)
    nxt = xb_ref[0].reshape(M, Cin)
    tail = jnp.zeros((Wpad, Cin), xa_ref.dtype)
    slab = jnp.concatenate([a, nxt, tail], axis=0)

    acc = jnp.zeros((M, Cout), jnp.float32)
    if fold_dy:
        for dx in range(3):
            s = slab[dx:]
            lhs = jnp.concatenate(
                [s[0:M], s[Wpad:Wpad + M], s[2 * Wpad:2 * Wpad + M]], axis=1)
            acc = acc + jnp.dot(lhs, w_ref[dx],
                                preferred_element_type=jnp.float32)
    else:
        slabs = (slab, slab[1:], slab[2:])
        for dy in range(3):
            base = dy * Wpad
            for dx in range(3):
                acc = acc + jnp.dot(slabs[dx][base:base + M], w_ref[dy, dx],
                                    preferred_element_type=jnp.float32)

    out = jnp.maximum(acc + b_ref[...], 0.0)
    out = out.reshape(TH, Wpad, Cout)[:, :W, :]
    o_ref[0] = out.astype(o_ref.dtype)


@functools.lru_cache(maxsize=None)
def _build_conv(N, H, W, Cin, Cout, TH, Wpad, fold_dy):
    body = functools.partial(_conv_kernel, TH=TH, W=W, Wpad=Wpad,
                             fold_dy=fold_dy)
    if fold_dy:
        w_spec = pl.BlockSpec((3, 3 * Cin, Cout), lambda n, r: (0, 0, 0))
    else:
        w_spec = pl.BlockSpec((3, 3, Cin, Cout), lambda n, r: (0, 0, 0, 0))
    return pl.pallas_call(
        body,
        out_shape=jax.ShapeDtypeStruct((N, H, W, Cout), jnp.bfloat16),
        grid=(N, H // TH),
        in_specs=[
            pl.BlockSpec((1, TH, Wpad, Cin), lambda n, r: (n, r, 0, 0)),
            pl.BlockSpec((1, TH, Wpad, Cin), lambda n, r: (n, r + 1, 0, 0)),
            w_spec,
            pl.BlockSpec((1, Cout), lambda n, r: (0, 0)),
        ],
        out_specs=pl.BlockSpec((1, TH, W, Cout), lambda n, r: (n, r, 0, 0)),
        compiler_params=pltpu.CompilerParams(
            dimension_semantics=("parallel", "parallel"),
            vmem_limit_bytes=_VMEM_LIMIT),
    )


def _conv3x3_relu(x, w, b):
    N, H, W, Cin = x.shape
    Cout = w.shape[-1]
    th_cap = 16 if max(Cin, Cout) <= 128 else (28 if Cout <= 256 else 28)
    TH = _pick_div(H, th_cap)
    Wpad = _round_up(W + 2, 16)
    xp = jnp.pad(x, ((0, 0), (1, TH - 1), (1, Wpad - W - 1), (0, 0)))
    fold_dy = Cin % 128 == 0
    if fold_dy:
        wk = w.transpose(1, 0, 2, 3).reshape(3, 3 * Cin, Cout)
    else:
        wk = w
    return _build_conv(N, H, W, Cin, Cout, TH, Wpad, fold_dy)(
        xp, xp, wk.astype(jnp.bfloat16), b)


def _pool_kernel(x_ref, o_ref, *, TR):
    v = x_ref[0]
    v = jnp.max(v, axis=2)
    v = v.reshape(TR, 2, v.shape[1], v.shape[2])
    o_ref[0] = jnp.max(v, axis=1)


@functools.lru_cache(maxsize=None)
def _build_pool(N, H, W, C, TR):
    Ho, Wo = H // 2, W // 2
    return pl.pallas_call(
        functools.partial(_pool_kernel, TR=TR),
        out_shape=jax.ShapeDtypeStruct((N, Ho, Wo, C), jnp.bfloat16),
        grid=(N, Ho // TR),
        in_specs=[pl.BlockSpec((1, 2 * TR, Wo, 2, C),
                               lambda n, r: (n, r, 0, 0, 0))],
        out_specs=pl.BlockSpec((1, TR, Wo, C), lambda n, r: (n, r, 0, 0)),
        compiler_params=pltpu.CompilerParams(
            dimension_semantics=("parallel", "parallel"),
            vmem_limit_bytes=_VMEM_LIMIT),
    )


def _maxpool2x2(x):
    N, H, W, C = x.shape
    Ho, Wo = H // 2, W // 2
    TR = _pick_div(Ho, 8)
    return _build_pool(N, H, W, C, TR)(x.reshape(N, H, Wo, 2, C))


def _l1_kernel(x_ref, y_ref, o_ref):
    @pl.when(pl.program_id(1) == 0)
    def _():
        o_ref[...] = jnp.zeros_like(o_ref)
    d = x_ref[...].astype(jnp.float32) - y_ref[...].astype(jnp.float32)
    o_ref[...] += jnp.sum(jnp.abs(d))


@functools.lru_cache(maxsize=None)
def _build_l1(Nh, Q, TQ):
    return pl.pallas_call(
        _l1_kernel,
        out_shape=jax.ShapeDtypeStruct((Nh, 8, 128), jnp.float32),
        grid=(Nh, Q // TQ),
        in_specs=[
            pl.BlockSpec((1, TQ, 512), lambda n, q: (n, q, 0)),
            pl.BlockSpec((1, TQ, 512), lambda n, q: (n + Nh, q, 0)),
        ],
        out_specs=pl.BlockSpec((1, 8, 128), lambda n, q: (n, 0, 0)),
        compiler_params=pltpu.CompilerParams(
            dimension_semantics=("parallel", "arbitrary"),
            vmem_limit_bytes=_VMEM_LIMIT),
    )


def _l1_mean(feat):
    N, H, W, C = feat.shape
    Nh = N // 2
    total = Nh * H * W * C
    Q = H * W * C // 512
    TQ = _pick_div(Q, 128)
    f2 = feat.reshape(N, Q, 512)
    s = _build_l1(Nh, Q, TQ)(f2, f2)
    return jnp.sum(s[:, 0, 0]) / jnp.float32(total)


def kernel(inp, tgt,
           w_0_0, b_0_0, w_0_1, b_0_1,
           w_1_0, b_1_0, w_1_1, b_1_1,
           w_2_0, b_2_0, w_2_1, b_2_1, w_2_2, b_2_2,
           w_3_0, b_3_0, w_3_1, b_3_1, w_3_2, b_3_2):
    params = (
        ((w_0_0, b_0_0), (w_0_1, b_0_1)),
        ((w_1_0, b_1_0), (w_1_1, b_1_1)),
        ((w_2_0, b_2_0), (w_2_1, b_2_1), (w_2_2, b_2_2)),
        ((w_3_0, b_3_0), (w_3_1, b_3_1), (w_3_2, b_3_2)),
    )
    both = jnp.concatenate([inp, tgt], axis=0)
    z = jnp.transpose(both, (0, 2, 3, 1)).astype(jnp.float32)
    z = ((z - _MEAN) / _STD).astype(jnp.bfloat16)

    loss = jnp.float32(0.0)
    for i, block in enumerate(params):
        if i > 0:
            z = _maxpool2x2(z)
        for (w, b) in block:
            z = _conv3x3_relu(z, w, b)
        loss = loss + _l1_mean(z)
    return loss


# speedup vs baseline: 1.2314x; 1.2314x over previous
import functools

import jax
import jax.numpy as jnp
from jax.experimental import pallas as pl
from jax.experimental.pallas import tpu as pltpu

_MEAN = jnp.array([0.485, 0.456, 0.406], jnp.float32)
_STD = jnp.array([0.229, 0.224, 0.225], jnp.float32)

_VMEM_LIMIT = 48 * 1024 * 1024


def _round_up(x, m):
    return (x + m - 1) // m * m


def _pick_div(n, cap):
    for t in range(min(n, cap), 0, -1):
        if n % t == 0:
            return t
    return 1


def _conv_taps(slab, w_ref, *, M, Wpad, Cout, fold_dy):
    acc = jnp.zeros((M, Cout), jnp.float32)
    if fold_dy:
        for dx in range(3):
            s = slab[dx:]
            lhs = jnp.concatenate(
                [s[0:M], s[Wpad:Wpad + M], s[2 * Wpad:2 * Wpad + M]], axis=1)
            acc = acc + jnp.dot(lhs, w_ref[dx],
                                preferred_element_type=jnp.float32)
    else:
        slabs = (slab, slab[1:], slab[2:])
        for dy in range(3):
            base = dy * Wpad
            for dx in range(3):
                acc = acc + jnp.dot(slabs[dx][base:base + M], w_ref[dy, dx],
                                    preferred_element_type=jnp.float32)
    return acc


def _make_slab(a, nxt, *, M, Wpad):
    Cin = a.shape[-1]
    tail = jnp.zeros((Wpad, Cin), a.dtype)
    return jnp.concatenate([a.reshape(M, Cin), nxt.reshape(M, Cin), tail],
                           axis=0)


def _conv_kernel(xa_ref, xb_ref, w_ref, b_ref, o_ref, *, TH, W, Wpad, fold_dy):
    Cin = xa_ref.shape[-1]
    Cout = o_ref.shape[-1]
    M = TH * Wpad
    slab = _make_slab(xa_ref[0], xb_ref[0], M=M, Wpad=Wpad)
    acc = _conv_taps(slab, w_ref, M=M, Wpad=Wpad, Cout=Cout, fold_dy=fold_dy)
    out = jnp.maximum(acc + b_ref[...], 0.0)
    out = out.reshape(TH, Wpad, Cout)[:, :W, :]
    o_ref[0] = out.astype(o_ref.dtype)


def _conv_kernel_merge(xa_ref, xb_ref, ya_ref, yb_ref, w_ref, b_ref, o_ref, *,
                       TH, W, Wpad, fold_dy, Nh):
    Cin = xa_ref.shape[-1]
    Cout = o_ref.shape[-1]
    M = TH * Wpad
    is_x = pl.program_id(0) < Nh
    a = jnp.where(is_x, xa_ref[0], ya_ref[0])
    nxt = jnp.where(is_x, xb_ref[0], yb_ref[0])
    slab = _make_slab(a, nxt, M=M, Wpad=Wpad)
    acc = _conv_taps(slab, w_ref, M=M, Wpad=Wpad, Cout=Cout, fold_dy=fold_dy)
    out = jnp.maximum(acc + b_ref[...], 0.0)
    out = out.reshape(TH, Wpad, Cout)[:, :W, :]
    o_ref[0] = out.astype(o_ref.dtype)


@functools.lru_cache(maxsize=None)
def _build_conv(N, H, W, Cin, Cout, TH, Wpad, fold_dy):
    body = functools.partial(_conv_kernel, TH=TH, W=W, Wpad=Wpad,
                             fold_dy=fold_dy)
    if fold_dy:
        w_spec = pl.BlockSpec((3, 3 * Cin, Cout), lambda n, r: (0, 0, 0))
    else:
        w_spec = pl.BlockSpec((3, 3, Cin, Cout), lambda n, r: (0, 0, 0, 0))
    return pl.pallas_call(
        body,
        out_shape=jax.ShapeDtypeStruct((N, H, W, Cout), jnp.bfloat16),
        grid=(N, H // TH),
        in_specs=[
            pl.BlockSpec((1, TH, Wpad, Cin), lambda n, r: (n, r, 0, 0)),
            pl.BlockSpec((1, TH, Wpad, Cin), lambda n, r: (n, r + 1, 0, 0)),
            w_spec,
            pl.BlockSpec((1, Cout), lambda n, r: (0, 0)),
        ],
        out_specs=pl.BlockSpec((1, TH, W, Cout), lambda n, r: (n, r, 0, 0)),
        compiler_params=pltpu.CompilerParams(
            dimension_semantics=("parallel", "parallel"),
            vmem_limit_bytes=_VMEM_LIMIT),
    )


@functools.lru_cache(maxsize=None)
def _build_conv_merge(Nh, H, W, Cin, Cout, TH, Wpad, fold_dy):
    body = functools.partial(_conv_kernel_merge, TH=TH, W=W, Wpad=Wpad,
                             fold_dy=fold_dy, Nh=Nh)
    if fold_dy:
        w_spec = pl.BlockSpec((3, 3 * Cin, Cout), lambda n, r: (0, 0, 0))
    else:
        w_spec = pl.BlockSpec((3, 3, Cin, Cout), lambda n, r: (0, 0, 0, 0))
    xi = lambda n, r: (jnp.minimum(n, Nh - 1), r, 0, 0)
    xj = lambda n, r: (jnp.minimum(n, Nh - 1), r + 1, 0, 0)
    yi = lambda n, r: (jnp.maximum(n - Nh, 0), r, 0, 0)
    yj = lambda n, r: (jnp.maximum(n - Nh, 0), r + 1, 0, 0)
    blk = (1, TH, Wpad, Cin)
    return pl.pallas_call(
        body,
        out_shape=jax.ShapeDtypeStruct((2 * Nh, H, W, Cout), jnp.bfloat16),
        grid=(2 * Nh, H // TH),
        in_specs=[
            pl.BlockSpec(blk, xi), pl.BlockSpec(blk, xj),
            pl.BlockSpec(blk, yi), pl.BlockSpec(blk, yj),
            w_spec,
            pl.BlockSpec((1, Cout), lambda n, r: (0, 0)),
        ],
        out_specs=pl.BlockSpec((1, TH, W, Cout), lambda n, r: (n, r, 0, 0)),
        compiler_params=pltpu.CompilerParams(
            dimension_semantics=("parallel", "parallel"),
            vmem_limit_bytes=_VMEM_LIMIT),
    )


def _conv_geom(H, W, Cin, Cout):
    th_cap = 16 if max(Cin, Cout) <= 128 else 28
    TH = _pick_div(H, th_cap)
    Wpad = _round_up(W + 2, 16)
    return TH, Wpad


def _conv_weights(w, Cin, Cout, fold_dy):
    if fold_dy:
        wk = w.transpose(1, 0, 2, 3).reshape(3, 3 * Cin, Cout)
    else:
        wk = w
    return wk.astype(jnp.bfloat16)


def _conv3x3_relu_merge(x, y, w, b):
    Nh, H, W, Cin = x.shape
    Cout = w.shape[-1]
    TH, Wpad = _conv_geom(H, W, Cin, Cout)
    pad = ((0, 0), (1, TH - 1), (1, Wpad - W - 1), (0, 0))
    xp = jnp.pad(x, pad)
    yp = jnp.pad(y, pad)
    fold_dy = Cin % 128 == 0
    wk = _conv_weights(w, Cin, Cout, fold_dy)
    return _build_conv_merge(Nh, H, W, Cin, Cout, TH, Wpad, fold_dy)(
        xp, xp, yp, yp, wk, b)


def _conv3x3_relu(x, w, b):
    N, H, W, Cin = x.shape
    Cout = w.shape[-1]
    TH, Wpad = _conv_geom(H, W, Cin, Cout)
    xp = jnp.pad(x, ((0, 0), (1, TH - 1), (1, Wpad - W - 1), (0, 0)))
    fold_dy = Cin % 128 == 0
    wk = _conv_weights(w, Cin, Cout, fold_dy)
    return _build_conv(N, H, W, Cin, Cout, TH, Wpad, fold_dy)(xp, xp, wk, b)


def _pool_kernel(x_ref, o_ref, *, TR):
    v = x_ref[0]
    v = jnp.max(v, axis=2)
    v = v.reshape(TR, 2, v.shape[1], v.shape[2])
    o_ref[0] = jnp.max(v, axis=1)


@functools.lru_cache(maxsize=None)
def _build_pool(N, H, W, C, TR):
    Ho, Wo = H // 2, W // 2
    return pl.pallas_call(
        functools.partial(_pool_kernel, TR=TR),
        out_shape=jax.ShapeDtypeStruct((N, Ho, Wo, C), jnp.bfloat16),
        grid=(N, Ho // TR),
        in_specs=[pl.BlockSpec((1, 2 * TR, Wo, 2, C),
                               lambda n, r: (n, r, 0, 0, 0))],
        out_specs=pl.BlockSpec((1, TR, Wo, C), lambda n, r: (n, r, 0, 0)),
        compiler_params=pltpu.CompilerParams(
            dimension_semantics=("parallel", "parallel"),
            vmem_limit_bytes=_VMEM_LIMIT),
    )


def _maxpool2x2(x):
    N, H, W, C = x.shape
    Ho, Wo = H // 2, W // 2
    TR = _pick_div(Ho, 8)
    return _build_pool(N, H, W, C, TR)(x.reshape(N, H, Wo, 2, C))


def _l1_kernel(x_ref, y_ref, o_ref):
    @pl.when(pl.program_id(1) == 0)
    def _():
        o_ref[...] = jnp.zeros_like(o_ref)
    d = x_ref[...].astype(jnp.float32) - y_ref[...].astype(jnp.float32)
    o_ref[...] += jnp.sum(jnp.abs(d))


@functools.lru_cache(maxsize=None)
def _build_l1(Nh, Q, TQ):
    return pl.pallas_call(
        _l1_kernel,
        out_shape=jax.ShapeDtypeStruct((Nh, 8, 128), jnp.float32),
        grid=(Nh, Q // TQ),
        in_specs=[
            pl.BlockSpec((1, TQ, 512), lambda n, q: (n, q, 0)),
            pl.BlockSpec((1, TQ, 512), lambda n, q: (n + Nh, q, 0)),
        ],
        out_specs=pl.BlockSpec((1, 8, 128), lambda n, q: (n, 0, 0)),
        compiler_params=pltpu.CompilerParams(
            dimension_semantics=("parallel", "arbitrary"),
            vmem_limit_bytes=_VMEM_LIMIT),
    )


def _l1_mean(feat):
    N, H, W, C = feat.shape
    Nh = N // 2
    total = Nh * H * W * C
    Q = H * W * C // 512
    TQ = _pick_div(Q, 128)
    f2 = feat.reshape(N, Q, 512)
    s = _build_l1(Nh, Q, TQ)(f2, f2)
    return jnp.sum(s[:, 0, 0]) / jnp.float32(total)


def kernel(inp, tgt,
           w_0_0, b_0_0, w_0_1, b_0_1,
           w_1_0, b_1_0, w_1_1, b_1_1,
           w_2_0, b_2_0, w_2_1, b_2_1, w_2_2, b_2_2,
           w_3_0, b_3_0, w_3_1, b_3_1, w_3_2, b_3_2):
    params = (
        ((w_0_0, b_0_0), (w_0_1, b_0_1)),
        ((w_1_0, b_1_0), (w_1_1, b_1_1)),
        ((w_2_0, b_2_0), (w_2_1, b_2_1), (w_2_2, b_2_2)),
        ((w_3_0, b_3_0), (w_3_1, b_3_1), (w_3_2, b_3_2)),
    )
    def _norm(v):
        v = jnp.transpose(v, (0, 2, 3, 1)).astype(jnp.float32)
        return ((v - _MEAN) / _STD).astype(jnp.bfloat16)

    x = _norm(inp)
    y = _norm(tgt)

    loss = jnp.float32(0.0)
    for i, block in enumerate(params):
        if i == 0:
            (w, b) = block[0]
            z = _conv3x3_relu_merge(x, y, w, b)
            rest = block[1:]
        else:
            z = _maxpool2x2(z)
            rest = block
        for (w, b) in rest:
            z = _conv3x3_relu(z, w, b)
        loss = loss + _l1_mean(z)
    return loss


# speedup vs baseline: 1.8900x; 1.5348x over previous
import functools

import jax
import jax.numpy as jnp
from jax import lax
from jax.experimental import pallas as pl
from jax.experimental.pallas import tpu as pltpu

_MEAN = jnp.array([0.485, 0.456, 0.406], jnp.float32)
_STD = jnp.array([0.229, 0.224, 0.225], jnp.float32)

_VMEM_LIMIT = 48 * 1024 * 1024


def _round_up(x, m):
    return (x + m - 1) // m * m


def _pick_div(n, cap):
    for t in range(min(n, cap), 0, -1):
        if n % t == 0:
            return t
    return 1


def _conv_body(x0, x1, x2, y0, y1, y2, w_ref, b_ref, *out_refs,
               TH, W, Wpad, Cout, R, mode, Wpad2):
    r = pl.program_id(1)
    Cin = x0.shape[-1]
    M = TH * Wpad
    Ms = M + 8
    dt = x0.dtype

    on_prev = jnp.where(r > 0, 1.0, 0.0).astype(dt)
    on_next = jnp.where(r < R - 1, 1.0, 0.0).astype(dt)

    def slab_of(a, b, c):
        prev = a[...].reshape(Wpad, Cin) * on_prev
        b = b[...].reshape(M, Cin)
        nxt = c[...].reshape(Wpad, Cin) * on_next
        tail = jnp.zeros((8, Cin), dt)
        return jnp.concatenate([prev, b, nxt, tail], axis=0)

    def lhs_of(s):
        return jnp.concatenate(
            [s[0:Ms], s[Wpad:Wpad + Ms], s[2 * Wpad:2 * Wpad + Ms]], axis=1)

    LHS = jnp.concatenate(
        [lhs_of(slab_of(x0, x1, x2)), lhs_of(slab_of(y0, y1, y2))], axis=0)
    T = jnp.dot(LHS, w_ref[...], preferred_element_type=jnp.float32)

    bias = b_ref[...]
    col = lax.broadcasted_iota(jnp.int32, (TH, Wpad, 1), 1)
    keep = (col >= 1) & (col <= W)

    def head(Tp):
        A = Tp[:, :Cout]
        B = Tp[:, Cout:2 * Cout]
        Cc = Tp[:, 2 * Cout:3 * Cout]
        Az = jnp.concatenate([jnp.zeros((8, Cout), jnp.float32), A], axis=0)
        o = Az[7:7 + M] + B[0:M] + Cc[1:M + 1]
        o = jnp.maximum(o + bias, 0.0).reshape(TH, Wpad, Cout)
        return jnp.where(keep, o, 0.0)

    ox = head(T[0:Ms])
    oy = head(T[Ms:2 * Ms])

    if mode == "mid":
        o_ref, = out_refs
        o_ref[0, 0] = ox.astype(jnp.bfloat16)
        o_ref[1, 0] = oy.astype(jnp.bfloat16)
        return

    if mode == "pool_l1":
        p_ref, l_ref = out_refs
    else:
        l_ref, = out_refs

    @pl.when(r == 0)
    def _():
        l_ref[...] = jnp.zeros_like(l_ref)
    l_ref[...] += jnp.sum(jnp.abs(ox - oy))

    if mode == "pool_l1":
        Wo = W // 2

        def pool(o):
            v = o[:, 1:W + 1, :].reshape(TH, Wo, 2, Cout)
            v = jnp.max(v, axis=2)
            v = v.reshape(TH // 2, 2, Wo, Cout)
            v = jnp.max(v, axis=1)
            z1 = jnp.zeros((TH // 2, 1, Cout), v.dtype)
            z2 = jnp.zeros((TH // 2, Wpad2 - Wo - 1, Cout), v.dtype)
            return jnp.concatenate([z1, v, z2], axis=1).astype(jnp.bfloat16)

        p_ref[0, 0] = pool(ox)
        p_ref[1, 0] = pool(oy)


@functools.lru_cache(maxsize=None)
def _build_conv(Nh, H, W, Cin, Cout, TH, Wpad, mode, split_in, Wpad2):
    R = H // TH
    body = functools.partial(_conv_body, TH=TH, W=W, Wpad=Wpad, Cout=Cout,
                             R=R, mode=mode, Wpad2=Wpad2)

    in_specs = []
    if split_in:
        for _s in range(2):
            in_specs.append(pl.BlockSpec(
                (1, 1, 1, Wpad, Cin),
                lambda n, r: (n, jnp.maximum(r - 1, 0), TH - 1, 0, 0)))
            in_specs.append(pl.BlockSpec(
                (1, 1, TH, Wpad, Cin), lambda n, r: (n, r, 0, 0, 0)))
            in_specs.append(pl.BlockSpec(
                (1, 1, 1, Wpad, Cin),
                lambda n, r: (n, jnp.minimum(r + 1, R - 1), 0, 0, 0)))
    else:
        for s in range(2):
            in_specs.append(pl.BlockSpec(
                (1, 1, 1, 1, Wpad, Cin),
                lambda n, r, s=s: (s, n, jnp.maximum(r - 1, 0), TH - 1, 0, 0)))
            in_specs.append(pl.BlockSpec(
                (1, 1, 1, TH, Wpad, Cin),
                lambda n, r, s=s: (s, n, r, 0, 0, 0)))
            in_specs.append(pl.BlockSpec(
                (1, 1, 1, 1, Wpad, Cin),
                lambda n, r, s=s: (s, n, jnp.minimum(r + 1, R - 1), 0, 0, 0)))
    in_specs.append(pl.BlockSpec((3 * Cin, max(3 * Cout, 256)),
                                 lambda n, r: (0, 0)))
    in_specs.append(pl.BlockSpec((1, Cout), lambda n, r: (0, 0)))

    feat_shape = jax.ShapeDtypeStruct((2, Nh, H, Wpad, Cout), jnp.bfloat16)
    feat_spec = pl.BlockSpec((2, 1, TH, Wpad, Cout), lambda n, r: (0, n, r, 0, 0))
    pool_shape = jax.ShapeDtypeStruct((2, Nh, H // 2, Wpad2, Cout), jnp.bfloat16)
    pool_spec = pl.BlockSpec((2, 1, TH // 2, Wpad2, Cout),
                             lambda n, r: (0, n, r, 0, 0))
    l1_shape = jax.ShapeDtypeStruct((Nh, 8, 128), jnp.float32)
    l1_spec = pl.BlockSpec((1, 8, 128), lambda n, r: (n, 0, 0))

    if mode == "mid":
        out_shape, out_specs = feat_shape, feat_spec
        sem = ("parallel", "parallel")
    elif mode == "pool_l1":
        out_shape = (pool_shape, l1_shape)
        out_specs = (pool_spec, l1_spec)
        sem = ("parallel", "arbitrary")
    else:
        out_shape, out_specs = l1_shape, l1_spec
        sem = ("parallel", "arbitrary")

    return pl.pallas_call(
        body,
        out_shape=out_shape,
        grid=(Nh, R),
        in_specs=in_specs,
        out_specs=out_specs,
        compiler_params=pltpu.CompilerParams(
            dimension_semantics=sem,
            vmem_limit_bytes=_VMEM_LIMIT),
    )


def _geom(H, W, Cout):
    th_cap = 32 if Cout <= 128 else 28
    TH = _pick_div(H, th_cap)
    if TH % 2 and TH < H:
        TH = _pick_div(H, th_cap - 1)
    Wpad = _round_up(W + 2, 16)
    return TH, Wpad


def _fold_w(w):
    KH, KW, Cin, Cout = w.shape
    wf = (w.transpose(0, 2, 1, 3).reshape(KH * Cin, KW * Cout)
          .astype(jnp.bfloat16))
    if KW * Cout < 256:
        wf = jnp.pad(wf, ((0, 0), (0, 256 - KW * Cout)))
    return wf


def kernel(inp, tgt,
           w_0_0, b_0_0, w_0_1, b_0_1,
           w_1_0, b_1_0, w_1_1, b_1_1,
           w_2_0, b_2_0, w_2_1, b_2_1, w_2_2, b_2_2,
           w_3_0, b_3_0, w_3_1, b_3_1, w_3_2, b_3_2):
    blocks = (
        ((w_0_0, b_0_0), (w_0_1, b_0_1)),
        ((w_1_0, b_1_0), (w_1_1, b_1_1)),
        ((w_2_0, b_2_0), (w_2_1, b_2_1), (w_2_2, b_2_2)),
        ((w_3_0, b_3_0), (w_3_1, b_3_1), (w_3_2, b_3_2)),
    )
    Nh, _, H, W = inp.shape

    TH0, Wpad0 = _geom(H, W, blocks[0][0][0].shape[-1])

    def _prep(v):
        v = jnp.transpose(v, (0, 2, 3, 1)).astype(jnp.float32)
        v = (v - _MEAN) / _STD
        v = jnp.pad(v, ((0, 0), (0, 0), (1, Wpad0 - W - 1), (0, 0)))
        return v.astype(jnp.bfloat16)

    z = (_prep(inp), _prep(tgt))

    loss = jnp.float32(0.0)
    h, w_cur = H, W
    for bi, block in enumerate(blocks):
        nconv = len(block)
        for li, (w, b) in enumerate(block):
            Cin = w.shape[2]
            Cout = w.shape[3]
            last = li == nconv - 1
            if not last:
                mode = "mid"
            elif bi < 3:
                mode = "pool_l1"
            else:
                mode = "l1"
            TH, Wpad = _geom(h, w_cur, Cout)
            Wpad2 = _round_up(w_cur // 2 + 2, 16) if mode == "pool_l1" else 16
            split_in = isinstance(z, tuple)
            call = _build_conv(Nh, h, w_cur, Cin, Cout, TH, Wpad, mode,
                               split_in, Wpad2)
            R = h // TH
            if split_in:
                zx = z[0].reshape(Nh, R, TH, Wpad, Cin)
                zy = z[1].reshape(Nh, R, TH, Wpad, Cin)
                args = [zx, zx, zx, zy, zy, zy]
            else:
                zr = z.reshape(2, Nh, R, TH, Wpad, Cin)
                args = [zr] * 6
            out = call(*args, _fold_w(w), b)
            if mode == "mid":
                z = out
            elif mode == "pool_l1":
                z, l1 = out
                total = Nh * h * w_cur * Cout
                loss = loss + jnp.sum(l1[:, 0, 0]) / jnp.float32(total)
                h, w_cur = h // 2, w_cur // 2
            else:
                l1 = out
                total = Nh * h * w_cur * Cout
                loss = loss + jnp.sum(l1[:, 0, 0]) / jnp.float32(total)
    return loss


# speedup vs baseline: 2.4429x; 1.2925x over previous
import functools

import jax
import jax.numpy as jnp
from jax import lax
from jax.experimental import pallas as pl
from jax.experimental.pallas import tpu as pltpu

_MEAN = jnp.array([0.485, 0.456, 0.406], jnp.float32)
_STD = jnp.array([0.229, 0.224, 0.225], jnp.float32)

_VMEM_LIMIT = 48 * 1024 * 1024


def _round_up(x, m):
    return (x + m - 1) // m * m


def _pick_div(n, cap):
    for t in range(min(n, cap), 0, -1):
        if n % t == 0:
            return t
    return 1


def _conv_body(x0, x1, x2, y0, y1, y2, w_ref, b_ref, *out_refs,
               TH, W, Cout, R, mode):
    r = pl.program_id(1)
    Cin = x0.shape[-1]
    M = TH * W
    Ms = M + 8
    dt = x0.dtype

    on_prev = jnp.where(r > 0, 1.0, 0.0).astype(dt)
    on_next = jnp.where(r < R - 1, 1.0, 0.0).astype(dt)

    def slab_of(a, b, c):
        prev = a[...].reshape(W, Cin) * on_prev
        b = b[...].reshape(M, Cin)
        nxt = c[...].reshape(W, Cin) * on_next
        tail = jnp.zeros((W + 8, Cin), dt)
        return jnp.concatenate([prev, b, nxt, tail], axis=0)

    def lhs_of(s):
        return jnp.concatenate(
            [s[0:Ms], s[W:W + Ms], s[2 * W:2 * W + Ms]], axis=1)

    LHS = jnp.concatenate(
        [lhs_of(slab_of(x0, x1, x2)), lhs_of(slab_of(y0, y1, y2))], axis=0)
    T = jnp.dot(LHS, w_ref[...], preferred_element_type=jnp.float32)

    bias = b_ref[...]
    col = lax.broadcasted_iota(jnp.int32, (TH, W, 1), 1)
    not_first = col >= 1
    not_last = col <= W - 2

    def head(Tp):
        A = Tp[:, :Cout]
        B = Tp[:, Cout:2 * Cout]
        Cc = Tp[:, 2 * Cout:3 * Cout]
        Az = jnp.concatenate([jnp.zeros((8, Cout), jnp.float32), A], axis=0)
        a3 = Az[7:7 + M].reshape(TH, W, Cout)
        b3 = B[0:M].reshape(TH, W, Cout)
        c3 = Cc[1:M + 1].reshape(TH, W, Cout)
        o = (jnp.where(not_first, a3, 0.0) + b3
             + jnp.where(not_last, c3, 0.0))
        return jnp.maximum(o + bias, 0.0)

    ox = head(T[0:Ms])
    oy = head(T[Ms:2 * Ms])

    if mode == "mid":
        o_ref, = out_refs
        o_ref[0, 0] = ox.astype(jnp.bfloat16)
        o_ref[1, 0] = oy.astype(jnp.bfloat16)
        return

    if mode == "pool_l1":
        p_ref, l_ref = out_refs
    else:
        l_ref, = out_refs

    @pl.when(r == 0)
    def _():
        l_ref[...] = jnp.zeros_like(l_ref)
    l_ref[...] += jnp.sum(jnp.abs(ox - oy))

    if mode == "pool_l1":
        Wo = W // 2

        def pool(o):
            v = o.reshape(TH, Wo, 2, Cout)
            v = jnp.max(v, axis=2)
            v = v.reshape(TH // 2, 2, Wo, Cout)
            v = jnp.max(v, axis=1)
            return v.astype(jnp.bfloat16)

        p_ref[0, 0] = pool(ox)
        p_ref[1, 0] = pool(oy)


@functools.lru_cache(maxsize=None)
def _build_conv(Nh, H, W, Cin, Cout, TH, mode, split_in):
    R = H // TH
    body = functools.partial(_conv_body, TH=TH, W=W, Cout=Cout,
                             R=R, mode=mode)

    in_specs = []
    if split_in:
        for _s in range(2):
            in_specs.append(pl.BlockSpec(
                (1, 1, 1, W, Cin),
                lambda n, r: (n, jnp.maximum(r - 1, 0), TH - 1, 0, 0)))
            in_specs.append(pl.BlockSpec(
                (1, 1, TH, W, Cin), lambda n, r: (n, r, 0, 0, 0)))
            in_specs.append(pl.BlockSpec(
                (1, 1, 1, W, Cin),
                lambda n, r: (n, jnp.minimum(r + 1, R - 1), 0, 0, 0)))
    else:
        for s in range(2):
            in_specs.append(pl.BlockSpec(
                (1, 1, 1, 1, W, Cin),
                lambda n, r, s=s: (s, n, jnp.maximum(r - 1, 0), TH - 1, 0, 0)))
            in_specs.append(pl.BlockSpec(
                (1, 1, 1, TH, W, Cin),
                lambda n, r, s=s: (s, n, r, 0, 0, 0)))
            in_specs.append(pl.BlockSpec(
                (1, 1, 1, 1, W, Cin),
                lambda n, r, s=s: (s, n, jnp.minimum(r + 1, R - 1), 0, 0, 0)))
    in_specs.append(pl.BlockSpec((3 * Cin, max(3 * Cout, 256)),
                                 lambda n, r: (0, 0)))
    in_specs.append(pl.BlockSpec((1, Cout), lambda n, r: (0, 0)))

    feat_shape = jax.ShapeDtypeStruct((2, Nh, H, W, Cout), jnp.bfloat16)
    feat_spec = pl.BlockSpec((2, 1, TH, W, Cout), lambda n, r: (0, n, r, 0, 0))
    pool_shape = jax.ShapeDtypeStruct((2, Nh, H // 2, W // 2, Cout),
                                      jnp.bfloat16)
    pool_spec = pl.BlockSpec((2, 1, TH // 2, W // 2, Cout),
                             lambda n, r: (0, n, r, 0, 0))
    l1_shape = jax.ShapeDtypeStruct((Nh, 8, 128), jnp.float32)
    l1_spec = pl.BlockSpec((1, 8, 128), lambda n, r: (n, 0, 0))

    if mode == "mid":
        out_shape, out_specs = feat_shape, feat_spec
        sem = ("parallel", "parallel")
    elif mode == "pool_l1":
        out_shape = (pool_shape, l1_shape)
        out_specs = (pool_spec, l1_spec)
        sem = ("parallel", "arbitrary")
    else:
        out_shape, out_specs = l1_shape, l1_spec
        sem = ("parallel", "arbitrary")

    return pl.pallas_call(
        body,
        out_shape=out_shape,
        grid=(Nh, R),
        in_specs=in_specs,
        out_specs=out_specs,
        compiler_params=pltpu.CompilerParams(
            dimension_semantics=sem,
            vmem_limit_bytes=_VMEM_LIMIT),
    )


def _geom(H, Cout):
    th_cap = 32 if Cout <= 128 else 28
    TH = _pick_div(H, th_cap)
    if TH % 2 and TH < H:
        TH = _pick_div(H, th_cap - 1)
    return TH


def _fold_w(w):
    KH, KW, Cin, Cout = w.shape
    wf = (w.transpose(0, 2, 1, 3).reshape(KH * Cin, KW * Cout)
          .astype(jnp.bfloat16))
    if KW * Cout < 256:
        wf = jnp.pad(wf, ((0, 0), (0, 256 - KW * Cout)))
    return wf


def kernel(inp, tgt,
           w_0_0, b_0_0, w_0_1, b_0_1,
           w_1_0, b_1_0, w_1_1, b_1_1,
           w_2_0, b_2_0, w_2_1, b_2_1, w_2_2, b_2_2,
           w_3_0, b_3_0, w_3_1, b_3_1, w_3_2, b_3_2):
    blocks = (
        ((w_0_0, b_0_0), (w_0_1, b_0_1)),
        ((w_1_0, b_1_0), (w_1_1, b_1_1)),
        ((w_2_0, b_2_0), (w_2_1, b_2_1), (w_2_2, b_2_2)),
        ((w_3_0, b_3_0), (w_3_1, b_3_1), (w_3_2, b_3_2)),
    )
    Nh, _, H, W = inp.shape

    def _prep(v):
        v = jnp.transpose(v, (0, 2, 3, 1)).astype(jnp.float32)
        v = (v - _MEAN) / _STD
        return v.astype(jnp.bfloat16)

    z = (_prep(inp), _prep(tgt))

    loss = jnp.float32(0.0)
    h, w_cur = H, W
    for bi, block in enumerate(blocks):
        nconv = len(block)
        for li, (w, b) in enumerate(block):
            Cin = w.shape[2]
            Cout = w.shape[3]
            last = li == nconv - 1
            if not last:
                mode = "mid"
            elif bi < 3:
                mode = "pool_l1"
            else:
                mode = "l1"
            TH = _geom(h, Cout)
            split_in = isinstance(z, tuple)
            call = _build_conv(Nh, h, w_cur, Cin, Cout, TH, mode, split_in)
            R = h // TH
            if split_in:
                zx = z[0].reshape(Nh, R, TH, w_cur, Cin)
                zy = z[1].reshape(Nh, R, TH, w_cur, Cin)
                args = [zx, zx, zx, zy, zy, zy]
            else:
                zr = z.reshape(2, Nh, R, TH, w_cur, Cin)
                args = [zr] * 6
            out = call(*args, _fold_w(w), b)
            if mode == "mid":
                z = out
            elif mode == "pool_l1":
                z, l1 = out
                total = Nh * h * w_cur * Cout
                loss = loss + jnp.sum(l1[:, 0, 0]) / jnp.float32(total)
                h, w_cur = h // 2, w_cur // 2
            else:
                l1 = out
                total = Nh * h * w_cur * Cout
                loss = loss + jnp.sum(l1[:, 0, 0]) / jnp.float32(total)
    return loss
